# baseline (speedup 1.0000x reference)
import functools

import jax
import jax.numpy as jnp
from jax import lax
from jax.experimental import pallas as pl
from jax.experimental.pallas import tpu as pltpu
from jax.experimental.pallas import tpu_sc as plsc

N = 2048
L = 16
NC = 2
NS = 16
NW = NC * NS
ROWS = N // NW
GROUPS = ROWS // L
G_CONST = 1.0


def _rsqrt(d2):
    i = plsc.bitcast(d2, jnp.int32)
    i = 0x5F3759DF - (i >> 1)
    y = plsc.bitcast(i, jnp.float32)
    h = 0.5 * d2
    y = y * (1.5 - h * y * y)
    y = y * (1.5 - h * y * y)
    return y


def _pair_acc(qxi, qyi, qxk, qyk, mk, ax, ay):
    dx = qxi - qxk
    dy = qyi - qyk
    d2 = jnp.maximum(dx * dx + dy * dy, 1e-24)
    y = _rsqrt(d2)
    w = y * y * y * mk
    return ax + w * dx, ay + w * dy


def _body(qx_h, qy_h, px_h, py_h, mm_h,
          opx_h, opy_h, ox_h, oy_h,
          qx_v, qy_v, mm_v, px_v, py_v, opx_v, opy_v, ox_v, oy_v):
    c = lax.axis_index("c")
    s = lax.axis_index("s")
    wid = s * NC + c
    base = wid * ROWS

    pltpu.sync_copy(qx_h, qx_v)
    pltpu.sync_copy(qy_h, qy_v)
    pltpu.sync_copy(mm_h, mm_v)
    pltpu.sync_copy(px_h.at[pl.ds(base, ROWS)], px_v)
    pltpu.sync_copy(py_h.at[pl.ds(base, ROWS)], py_v)

    lanes = lax.iota(jnp.int32, L)
    zero = jnp.zeros((L,), jnp.float32)

    qxi = [qx_v[pl.ds(base + g * L, L)] for g in range(GROUPS)]
    qyi = [qy_v[pl.ds(base + g * L, L)] for g in range(GROUPS)]

    def kbody(k, carry):
        idx = jnp.full((L,), k, jnp.int32)
        qxk = plsc.load_gather(qx_v, [idx])
        qyk = plsc.load_gather(qy_v, [idx])
        mk = plsc.load_gather(mm_v, [idx])
        out = []
        for g in range(GROUPS):
            ax, ay = _pair_acc(qxi[g], qyi[g], qxk, qyk, mk,
                               carry[2 * g], carry[2 * g + 1])
            out += [ax, ay]
        return tuple(out)

    accs = lax.fori_loop(0, N, kbody, (zero,) * (2 * GROUPS))
    accs = list(accs)

    for g in range(GROUPS):
        gi = lanes + (base + g * L)
        for nidx in (jnp.maximum(gi - 1, 0), jnp.minimum(gi + 1, N - 1)):
            qxn = plsc.load_gather(qx_v, [nidx])
            qyn = plsc.load_gather(qy_v, [nidx])
            mn = plsc.load_gather(mm_v, [nidx])
            ax, ay = _pair_acc(qxi[g], qyi[g], qxn, qyn, mn,
                               accs[2 * g], accs[2 * g + 1])
            accs[2 * g], accs[2 * g + 1] = ax, ay

    zidx = jnp.zeros((L,), jnp.int32)
    m0 = plsc.load_gather(mm_v, [zidx])
    inv_m0 = 1.0 / m0
    for g in range(GROUPS):
        mi = mm_v[pl.ds(base + g * L, L)]
        sl = pl.ds(g * L, L)
        ox_v[sl] = accs[2 * g] * (G_CONST * mi)
        oy_v[sl] = accs[2 * g + 1] * (G_CONST * mi)
        opx_v[sl] = px_v[sl] * inv_m0
        opy_v[sl] = py_v[sl] * inv_m0

    pltpu.sync_copy(opx_v, opx_h.at[pl.ds(base, ROWS)])
    pltpu.sync_copy(opy_v, opy_h.at[pl.ds(base, ROWS)])
    pltpu.sync_copy(ox_v, ox_h.at[pl.ds(base, ROWS)])
    pltpu.sync_copy(oy_v, oy_h.at[pl.ds(base, ROWS)])


_f32 = jnp.float32
_vec = jax.ShapeDtypeStruct((N,), _f32)

_nbody_sc = functools.partial(
    pl.kernel,
    out_type=(_vec, _vec, _vec, _vec),
    mesh=plsc.VectorSubcoreMesh(core_axis_name="c", subcore_axis_name="s"),
    compiler_params=pltpu.CompilerParams(needs_layout_passes=False),
    scratch_types=[
        pltpu.VMEM((N,), _f32),
        pltpu.VMEM((N,), _f32),
        pltpu.VMEM((N,), _f32),
        pltpu.VMEM((ROWS,), _f32),
        pltpu.VMEM((ROWS,), _f32),
        pltpu.VMEM((ROWS,), _f32),
        pltpu.VMEM((ROWS,), _f32),
        pltpu.VMEM((ROWS,), _f32),
        pltpu.VMEM((ROWS,), _f32),
    ],
)(_body)


@jax.jit
def _run(x, m):
    qx = x[:, 0]
    qy = x[:, 1]
    px = x[:, 2]
    py = x[:, 3]
    mm = m[:, 0]
    opx, opy, ox, oy = _nbody_sc(qx, qy, px, py, mm)
    return jnp.stack([opx, opy, ox, oy], axis=-1)


def kernel(t, x, m):
    del t
    return _run(x, m)


# speedup vs baseline: 1.0014x; 1.0014x over previous
import functools

import jax
import jax.numpy as jnp
from jax import lax
from jax.experimental import pallas as pl
from jax.experimental.pallas import tpu as pltpu
from jax.experimental.pallas import tpu_sc as plsc

N = 2048
L = 16
NC = 2
NS = 16
NW = NC * NS
ROWS = N // NW
GROUPS = ROWS // L
G_CONST = 1.0


def _rsqrt(d2):
    i = plsc.bitcast(d2, jnp.int32)
    i = 0x5F3759DF - (i >> 1)
    y = plsc.bitcast(i, jnp.float32)
    u = d2 * (y * y)
    v = 1.25 - 0.375 * u
    s = 1.875 - u * v
    return y * s


def _pair_acc(qxi, qyi, qxk, qyk, mk, ax, ay):
    dx = qxi - qxk
    dy = qyi - qyk
    d2 = jnp.maximum(dx * dx + dy * dy, 1e-24)
    y = _rsqrt(d2)
    w = y * y * y * mk
    return ax + w * dx, ay + w * dy


def _body(qx_h, qy_h, px_h, py_h, mm_h,
          opx_h, opy_h, ox_h, oy_h,
          qx_v, qy_v, mm_v, px_v, py_v, opx_v, opy_v, ox_v, oy_v):
    c = lax.axis_index("c")
    s = lax.axis_index("s")
    wid = s * NC + c
    base = wid * ROWS

    pltpu.sync_copy(qx_h, qx_v)
    pltpu.sync_copy(qy_h, qy_v)
    pltpu.sync_copy(mm_h, mm_v)
    pltpu.sync_copy(px_h.at[pl.ds(base, ROWS)], px_v)
    pltpu.sync_copy(py_h.at[pl.ds(base, ROWS)], py_v)

    lanes = lax.iota(jnp.int32, L)
    zero = jnp.zeros((L,), jnp.float32)

    qxi = [qx_v[pl.ds(base + g * L, L)] for g in range(GROUPS)]
    qyi = [qy_v[pl.ds(base + g * L, L)] for g in range(GROUPS)]

    UNROLL = 4

    def kbody(k0, carry):
        out = carry
        for u in range(UNROLL):
            idx = jnp.full((L,), k0 * UNROLL + u, jnp.int32)
            qxk = plsc.load_gather(qx_v, [idx])
            qyk = plsc.load_gather(qy_v, [idx])
            mk = plsc.load_gather(mm_v, [idx])
            nxt = []
            for g in range(GROUPS):
                ax, ay = _pair_acc(qxi[g], qyi[g], qxk, qyk, mk,
                                   out[2 * g], out[2 * g + 1])
                nxt += [ax, ay]
            out = tuple(nxt)
        return out

    accs = lax.fori_loop(0, N // UNROLL, kbody, (zero,) * (2 * GROUPS))
    accs = list(accs)

    for g in range(GROUPS):
        gi = lanes + (base + g * L)
        for nidx in (jnp.maximum(gi - 1, 0), jnp.minimum(gi + 1, N - 1)):
            qxn = plsc.load_gather(qx_v, [nidx])
            qyn = plsc.load_gather(qy_v, [nidx])
            mn = plsc.load_gather(mm_v, [nidx])
            ax, ay = _pair_acc(qxi[g], qyi[g], qxn, qyn, mn,
                               accs[2 * g], accs[2 * g + 1])
            accs[2 * g], accs[2 * g + 1] = ax, ay

    zidx = jnp.zeros((L,), jnp.int32)
    m0 = plsc.load_gather(mm_v, [zidx])
    inv_m0 = 1.0 / m0
    for g in range(GROUPS):
        mi = mm_v[pl.ds(base + g * L, L)]
        sl = pl.ds(g * L, L)
        ox_v[sl] = accs[2 * g] * (G_CONST * mi)
        oy_v[sl] = accs[2 * g + 1] * (G_CONST * mi)
        opx_v[sl] = px_v[sl] * inv_m0
        opy_v[sl] = py_v[sl] * inv_m0

    pltpu.sync_copy(opx_v, opx_h.at[pl.ds(base, ROWS)])
    pltpu.sync_copy(opy_v, opy_h.at[pl.ds(base, ROWS)])
    pltpu.sync_copy(ox_v, ox_h.at[pl.ds(base, ROWS)])
    pltpu.sync_copy(oy_v, oy_h.at[pl.ds(base, ROWS)])


_f32 = jnp.float32
_vec = jax.ShapeDtypeStruct((N,), _f32)

_nbody_sc = functools.partial(
    pl.kernel,
    out_type=(_vec, _vec, _vec, _vec),
    mesh=plsc.VectorSubcoreMesh(core_axis_name="c", subcore_axis_name="s"),
    compiler_params=pltpu.CompilerParams(needs_layout_passes=False),
    scratch_types=[
        pltpu.VMEM((N,), _f32),
        pltpu.VMEM((N,), _f32),
        pltpu.VMEM((N,), _f32),
        pltpu.VMEM((ROWS,), _f32),
        pltpu.VMEM((ROWS,), _f32),
        pltpu.VMEM((ROWS,), _f32),
        pltpu.VMEM((ROWS,), _f32),
        pltpu.VMEM((ROWS,), _f32),
        pltpu.VMEM((ROWS,), _f32),
    ],
)(_body)


@jax.jit
def _run(x, m):
    qx = x[:, 0]
    qy = x[:, 1]
    px = x[:, 2]
    py = x[:, 3]
    mm = m[:, 0]
    opx, opy, ox, oy = _nbody_sc(qx, qy, px, py, mm)
    return jnp.stack([opx, opy, ox, oy], axis=-1)


def kernel(t, x, m):
    del t
    return _run(x, m)


# speedup vs baseline: 1.3294x; 1.3275x over previous
import functools

import jax
import jax.numpy as jnp
from jax import lax
from jax.experimental import pallas as pl
from jax.experimental.pallas import tpu as pltpu
from jax.experimental.pallas import tpu_sc as plsc

N = 2048
L = 16
NC = 2
NS = 16
NW = NC * NS
T_SC = 512
S_TC = N - T_SC
ROWS = T_SC // NW
BI = 256
G_CONST = 1.0


def _rsqrt_sc(d2):
    i = plsc.bitcast(d2, jnp.int32)
    i = 0x5F3759DF - (i >> 1)
    y = plsc.bitcast(i, jnp.float32)
    u = d2 * (y * y)
    v = 1.25 - 0.375 * u
    s = 1.875 - u * v
    return y * s


def _pair_acc(qxi, qyi, qxk, qyk, mk, ax, ay):
    dx = qxi - qxk
    dy = qyi - qyk
    d2 = jnp.maximum(dx * dx + dy * dy, 1e-24)
    y = _rsqrt_sc(d2)
    w = y * y * y * mk
    return ax + w * dx, ay + w * dy


def _lane_bcast(vec, j):
    return jnp.take_along_axis(vec, jnp.full((L,), j, jnp.int32), axis=0,
                               mode="promise_in_bounds")


def _sc_body(qx_h, qy_h, px_h, py_h, mm_h,
             opx_h, opy_h, ox_h, oy_h,
             qx_v, qy_v, mm_v, px_v, py_v, opx_v, opy_v, ox_v, oy_v):
    c = lax.axis_index("c")
    s = lax.axis_index("s")
    wid = s * NC + c
    base = S_TC + wid * ROWS
    obase = wid * ROWS

    pltpu.sync_copy(qx_h, qx_v)
    pltpu.sync_copy(qy_h, qy_v)
    pltpu.sync_copy(mm_h, mm_v)
    pltpu.sync_copy(px_h.at[pl.ds(base, ROWS)], px_v)
    pltpu.sync_copy(py_h.at[pl.ds(base, ROWS)], py_v)

    lanes = lax.iota(jnp.int32, L)
    zero = jnp.zeros((L,), jnp.float32)

    qxi = qx_v[pl.ds(base, L)]
    qyi = qy_v[pl.ds(base, L)]

    def cbody(ci, carry):
        ax, ay = carry
        sl = pl.ds(ci * L, L)
        qxc = qx_v[sl]
        qyc = qy_v[sl]
        mc = mm_v[sl]
        for j in range(L):
            qxk = _lane_bcast(qxc, j)
            qyk = _lane_bcast(qyc, j)
            mk = _lane_bcast(mc, j)
            ax, ay = _pair_acc(qxi, qyi, qxk, qyk, mk, ax, ay)
        return ax, ay

    ax, ay = lax.fori_loop(0, N // L, cbody, (zero, zero))

    gi = lanes + base
    for nidx in (gi - 1, jnp.minimum(gi + 1, N - 1)):
        qxn = plsc.load_gather(qx_v, [nidx])
        qyn = plsc.load_gather(qy_v, [nidx])
        mn = plsc.load_gather(mm_v, [nidx])
        ax, ay = _pair_acc(qxi, qyi, qxn, qyn, mn, ax, ay)

    m0 = plsc.load_gather(mm_v, [jnp.zeros((L,), jnp.int32)])
    inv_m0 = 1.0 / m0
    mi = mm_v[pl.ds(base, L)]
    ox_v[...] = ax * (G_CONST * mi)
    oy_v[...] = ay * (G_CONST * mi)
    opx_v[...] = px_v[...] * inv_m0
    opy_v[...] = py_v[...] * inv_m0

    pltpu.sync_copy(opx_v, opx_h.at[pl.ds(obase, ROWS)])
    pltpu.sync_copy(opy_v, opy_h.at[pl.ds(obase, ROWS)])
    pltpu.sync_copy(ox_v, ox_h.at[pl.ds(obase, ROWS)])
    pltpu.sync_copy(oy_v, oy_h.at[pl.ds(obase, ROWS)])


_f32 = jnp.float32
_svec = jax.ShapeDtypeStruct((T_SC,), _f32)

_nbody_sc = functools.partial(
    pl.kernel,
    out_type=(_svec, _svec, _svec, _svec),
    mesh=plsc.VectorSubcoreMesh(core_axis_name="c", subcore_axis_name="s"),
    compiler_params=pltpu.CompilerParams(needs_layout_passes=False),
    scratch_types=[
        pltpu.VMEM((N,), _f32),
        pltpu.VMEM((N,), _f32),
        pltpu.VMEM((N,), _f32),
        pltpu.VMEM((ROWS,), _f32),
        pltpu.VMEM((ROWS,), _f32),
        pltpu.VMEM((ROWS,), _f32),
        pltpu.VMEM((ROWS,), _f32),
        pltpu.VMEM((ROWS,), _f32),
        pltpu.VMEM((ROWS,), _f32),
    ],
)(_sc_body)


def _tc_body(qxc, qyc, mmc, qxr, qyr, mmr,
             qxl, qyl, mml, qxg, qyg, mmg,
             pxc, pyc, out_ref):
    qix = qxc[...]
    qiy = qyc[...]
    dx = qix - qxr[...]
    dy = qiy - qyr[...]
    d2 = jnp.maximum(dx * dx + dy * dy, 1e-24)
    y = lax.rsqrt(d2)
    w = y * y * y * mmr[...]
    accx = jnp.sum(w * dx, axis=1, keepdims=True)
    accy = jnp.sum(w * dy, axis=1, keepdims=True)
    for nx, ny, nm in ((qxl, qyl, mml), (qxg, qyg, mmg)):
        ddx = qix - nx[...]
        ddy = qiy - ny[...]
        dd2 = jnp.maximum(ddx * ddx + ddy * ddy, 1e-24)
        yy = lax.rsqrt(dd2)
        ww = yy * yy * yy * nm[...]
        accx = accx + ww * ddx
        accy = accy + ww * ddy
    mi = mmc[...] * G_CONST
    inv0 = 1.0 / mmr[0, 0]
    out_ref[...] = jnp.concatenate(
        [pxc[...] * inv0, pyc[...] * inv0, accx * mi, accy * mi], axis=1)


_col = pl.BlockSpec((BI, 1), lambda i: (i, 0))
_row = pl.BlockSpec((1, N), lambda i: (0, 0))

_nbody_tc = pl.pallas_call(
    _tc_body,
    grid=(S_TC // BI,),
    in_specs=[_col, _col, _col, _row, _row, _row,
              _col, _col, _col, _col, _col, _col,
              _col, _col],
    out_specs=pl.BlockSpec((BI, 4), lambda i: (i, 0)),
    out_shape=jax.ShapeDtypeStruct((S_TC, 4), _f32),
)


@jax.jit
def _run(x, m):
    qx = x[:, 0]
    qy = x[:, 1]
    px = x[:, 2]
    py = x[:, 3]
    mm = m[:, 0]

    qxl = jnp.concatenate([qx[:1], qx[:-1]])
    qyl = jnp.concatenate([qy[:1], qy[:-1]])
    mml = jnp.concatenate([mm[:1], mm[:-1]])
    qxg = jnp.concatenate([qx[1:], qx[-1:]])
    qyg = jnp.concatenate([qy[1:], qy[-1:]])
    mmg = jnp.concatenate([mm[1:], mm[-1:]])

    col = lambda a: a[:S_TC, None]
    row = lambda a: a[None, :]
    tc_out = _nbody_tc(col(qx), col(qy), col(mm),
                       row(qx), row(qy), row(mm),
                       col(qxl), col(qyl), col(mml),
                       col(qxg), col(qyg), col(mmg),
                       col(px), col(py))

    opx, opy, ox, oy = _nbody_sc(qx, qy, px, py, mm)
    sc_out = jnp.stack([opx, opy, ox, oy], axis=-1)
    return jnp.concatenate([tc_out, sc_out], axis=0)


def kernel(t, x, m):
    del t
    return _run(x, m)


# speedup vs baseline: 1.5391x; 1.1577x over previous
import functools

import jax
import jax.numpy as jnp
from jax import lax
from jax.experimental import pallas as pl
from jax.experimental.pallas import tpu as pltpu
from jax.experimental.pallas import tpu_sc as plsc

N = 2048
L = 16
NC = 2
NS = 16
NW = NC * NS
T_SC = 512
S_TC = N - T_SC
ROWS = T_SC // NW
BI = 256
G_CONST = 1.0


def _rsqrt_sc(d2):
    i = plsc.bitcast(d2, jnp.int32)
    i = 0x5F3759DF - (i >> 1)
    y = plsc.bitcast(i, jnp.float32)
    u = d2 * (y * y)
    v = 1.25 - 0.375 * u
    s = 1.875 - u * v
    return y * s


def _pair_acc(qxi, qyi, qxk, qyk, mk, ax, ay):
    dx = qxi - qxk
    dy = qyi - qyk
    d2 = jnp.maximum(dx * dx + dy * dy, 1e-24)
    y = _rsqrt_sc(d2)
    w = y * y * y * mk
    return ax + w * dx, ay + w * dy


def _lane_bcast(vec, j):
    return jnp.take_along_axis(vec, jnp.full((L,), j, jnp.int32), axis=0,
                               mode="promise_in_bounds")


def _sc_body(xt_h, mm_h,
             opx_h, opy_h, ox_h, oy_h,
             qx_v, qy_v, mm_v, px_v, py_v, opx_v, opy_v, ox_v, oy_v):
    c = lax.axis_index("c")
    s = lax.axis_index("s")
    wid = s * NC + c
    base = S_TC + wid * ROWS
    obase = wid * ROWS

    pltpu.sync_copy(xt_h.at[0], qx_v)
    pltpu.sync_copy(xt_h.at[1], qy_v)
    pltpu.sync_copy(mm_h, mm_v)
    pltpu.sync_copy(xt_h.at[2, pl.ds(base, ROWS)], px_v)
    pltpu.sync_copy(xt_h.at[3, pl.ds(base, ROWS)], py_v)

    lanes = lax.iota(jnp.int32, L)
    zero = jnp.zeros((L,), jnp.float32)

    qxi = qx_v[pl.ds(base, L)]
    qyi = qy_v[pl.ds(base, L)]

    def cbody(ci, carry):
        ax, ay = carry
        sl = pl.ds(ci * L, L)
        qxc = qx_v[sl]
        qyc = qy_v[sl]
        mc = mm_v[sl]
        for j in range(L):
            qxk = _lane_bcast(qxc, j)
            qyk = _lane_bcast(qyc, j)
            mk = _lane_bcast(mc, j)
            ax, ay = _pair_acc(qxi, qyi, qxk, qyk, mk, ax, ay)
        return ax, ay

    ax, ay = lax.fori_loop(0, N // L, cbody, (zero, zero))

    gi = lanes + base
    for nidx in (gi - 1, jnp.minimum(gi + 1, N - 1)):
        qxn = plsc.load_gather(qx_v, [nidx])
        qyn = plsc.load_gather(qy_v, [nidx])
        mn = plsc.load_gather(mm_v, [nidx])
        ax, ay = _pair_acc(qxi, qyi, qxn, qyn, mn, ax, ay)

    m0 = plsc.load_gather(mm_v, [jnp.zeros((L,), jnp.int32)])
    inv_m0 = 1.0 / m0
    mi = mm_v[pl.ds(base, L)]
    ox_v[...] = ax * (G_CONST * mi)
    oy_v[...] = ay * (G_CONST * mi)
    opx_v[...] = px_v[...] * inv_m0
    opy_v[...] = py_v[...] * inv_m0

    pltpu.sync_copy(opx_v, opx_h.at[pl.ds(obase, ROWS)])
    pltpu.sync_copy(opy_v, opy_h.at[pl.ds(obase, ROWS)])
    pltpu.sync_copy(ox_v, ox_h.at[pl.ds(obase, ROWS)])
    pltpu.sync_copy(oy_v, oy_h.at[pl.ds(obase, ROWS)])


_f32 = jnp.float32
_svec = jax.ShapeDtypeStruct((T_SC,), _f32)

_nbody_sc = functools.partial(
    pl.kernel,
    out_type=(_svec, _svec, _svec, _svec),
    mesh=plsc.VectorSubcoreMesh(core_axis_name="c", subcore_axis_name="s"),
    compiler_params=pltpu.CompilerParams(needs_layout_passes=False),
    scratch_types=[
        pltpu.VMEM((N,), _f32),
        pltpu.VMEM((N,), _f32),
        pltpu.VMEM((N,), _f32),
        pltpu.VMEM((ROWS,), _f32),
        pltpu.VMEM((ROWS,), _f32),
        pltpu.VMEM((ROWS,), _f32),
        pltpu.VMEM((ROWS,), _f32),
        pltpu.VMEM((ROWS,), _f32),
        pltpu.VMEM((ROWS,), _f32),
    ],
)(_sc_body)


def _tc_body(xb_ref, xf_ref, xt_ref, mc_ref, mf_ref, mr_ref, out_ref):
    i0 = pl.program_id(0) * BI
    xb = xb_ref[...]
    qix = xb[:, 0:1]
    qiy = xb[:, 1:2]
    qkx = xt_ref[0:1, :]
    qky = xt_ref[1:2, :]
    mk = mr_ref[...]
    dx = qix - qkx
    dy = qiy - qky
    d2 = jnp.maximum(dx * dx + dy * dy, 1e-24)
    y = lax.rsqrt(d2)
    w = y * y * y * mk
    accx = jnp.sum(w * dx, axis=1, keepdims=True)
    accy = jnp.sum(w * dy, axis=1, keepdims=True)

    prow = xf_ref[pl.ds(jnp.maximum(i0 - 1, 0), 1), :]
    nrow = xf_ref[pl.ds(jnp.minimum(i0 + BI, N - 1), 1), :]
    pm = mf_ref[pl.ds(jnp.maximum(i0 - 1, 0), 1), :]
    nm = mf_ref[pl.ds(jnp.minimum(i0 + BI, N - 1), 1), :]
    mcol = mc_ref[...]
    prev_x = jnp.concatenate([prow, xb[: BI - 1, :]], axis=0)
    next_x = jnp.concatenate([xb[1:, :], nrow], axis=0)
    prev_m = jnp.concatenate([pm, mcol[: BI - 1, :]], axis=0)
    next_m = jnp.concatenate([mcol[1:, :], nm], axis=0)
    for xn, mn in ((prev_x, prev_m), (next_x, next_m)):
        ddx = qix - xn[:, 0:1]
        ddy = qiy - xn[:, 1:2]
        dd2 = jnp.maximum(ddx * ddx + ddy * ddy, 1e-24)
        yy = lax.rsqrt(dd2)
        ww = yy * yy * yy * mn
        accx = accx + ww * ddx
        accy = accy + ww * ddy

    mi = mcol * G_CONST
    inv0 = 1.0 / mr_ref[0, 0]
    out_ref[...] = jnp.concatenate(
        [xb[:, 2:3] * inv0, xb[:, 3:4] * inv0, accx * mi, accy * mi], axis=1)


_nbody_tc = pl.pallas_call(
    _tc_body,
    grid=(S_TC // BI,),
    in_specs=[
        pl.BlockSpec((BI, 4), lambda i: (i, 0)),
        pl.BlockSpec((N, 4), lambda i: (0, 0)),
        pl.BlockSpec((4, N), lambda i: (0, 0)),
        pl.BlockSpec((BI, 1), lambda i: (i, 0)),
        pl.BlockSpec((N, 1), lambda i: (0, 0)),
        pl.BlockSpec((1, N), lambda i: (0, 0)),
    ],
    out_specs=pl.BlockSpec((BI, 4), lambda i: (i, 0)),
    out_shape=jax.ShapeDtypeStruct((S_TC, 4), _f32),
)


@jax.jit
def _run(x, m):
    xt = x.T
    mm = m.reshape(N)
    mr = m.reshape(1, N)

    tc_out = _nbody_tc(x, x, xt, m, m, mr)
    opx, opy, ox, oy = _nbody_sc(xt, mm)
    sc_out = jnp.stack([opx, opy, ox, oy], axis=-1)
    return jnp.concatenate([tc_out, sc_out], axis=0)


def kernel(t, x, m):
    del t
    return _run(x, m)


# speedup vs baseline: 1.8470x; 1.2000x over previous
import functools

import jax
import jax.numpy as jnp
from jax import lax
from jax.experimental import pallas as pl
from jax.experimental.pallas import tpu as pltpu
from jax.experimental.pallas import tpu_sc as plsc

N = 2048
L = 16
NC = 2
NS = 16
T_SC = 256
S_TC = N - T_SC
ROWS = 16
NG = T_SC // ROWS
GPC = NG // NC
HALF = N // 2
BI = 256
G_CONST = 1.0


def _rsqrt_sc(d2):
    i = plsc.bitcast(d2, jnp.int32)
    i = 0x5F3759DF - (i >> 1)
    y = plsc.bitcast(i, jnp.float32)
    u = d2 * (y * y)
    v = 1.25 - 0.375 * u
    s = 1.875 - u * v
    return y * s


def _pair_acc(qxi, qyi, qxk, qyk, mk, ax, ay):
    dx = qxi - qxk
    dy = qyi - qyk
    d2 = jnp.maximum(dx * dx + dy * dy, 1e-24)
    y = _rsqrt_sc(d2)
    w = y * y * y * mk
    return ax + w * dx, ay + w * dy


def _lane_bcast(vec, j):
    return jnp.take_along_axis(vec, jnp.full((L,), j, jnp.int32), axis=0,
                               mode="promise_in_bounds")


def _sc_body(xt_h, mm_h,
             opx_h, opy_h, ox_h, oy_h,
             qx_v, qy_v, mm_v, px_v, py_v, opx_v, opy_v, ox_v, oy_v,
             ax_st, ay_st, bx_v, by_v, shax, shay):
    c = lax.axis_index("c")
    s = lax.axis_index("s")
    g_local = s % GPC
    half = s // GPC
    group = c * GPC + g_local
    base = S_TC + group * ROWS
    obase = group * ROWS

    pltpu.sync_copy(xt_h.at[0], qx_v)
    pltpu.sync_copy(xt_h.at[1], qy_v)
    pltpu.sync_copy(mm_h, mm_v)
    pltpu.sync_copy(xt_h.at[2, pl.ds(base, ROWS)], px_v)
    pltpu.sync_copy(xt_h.at[3, pl.ds(base, ROWS)], py_v)

    lanes = lax.iota(jnp.int32, L)
    zero = jnp.zeros((L,), jnp.float32)

    qxi = qx_v[pl.ds(base, L)]
    qyi = qy_v[pl.ds(base, L)]

    def cbody(ci, carry):
        ax, ay = carry
        sl = pl.ds(ci * L, L)
        qxc = qx_v[sl]
        qyc = qy_v[sl]
        mc = mm_v[sl]
        for j in range(L):
            qxk = _lane_bcast(qxc, j)
            qyk = _lane_bcast(qyc, j)
            mk = _lane_bcast(mc, j)
            ax, ay = _pair_acc(qxi, qyi, qxk, qyk, mk, ax, ay)
        return ax, ay

    c0 = half * (HALF // L)
    ax, ay = lax.fori_loop(c0, c0 + HALF // L, cbody, (zero, zero))

    ax_st[...] = ax
    ay_st[...] = ay
    pltpu.sync_copy(ax_st, shax.at[g_local, half])
    pltpu.sync_copy(ay_st, shay.at[g_local, half])
    plsc.subcore_barrier()

    @pl.when(half == 0)
    def _():
        pltpu.sync_copy(shax.at[g_local, 1], bx_v)
        pltpu.sync_copy(shay.at[g_local, 1], by_v)
        axt = ax + bx_v[...]
        ayt = ay + by_v[...]

        gi = lanes + base
        a2x, a2y = axt, ayt
        for nidx in (gi - 1, jnp.minimum(gi + 1, N - 1)):
            qxn = plsc.load_gather(qx_v, [nidx])
            qyn = plsc.load_gather(qy_v, [nidx])
            mn = plsc.load_gather(mm_v, [nidx])
            a2x, a2y = _pair_acc(qxi, qyi, qxn, qyn, mn, a2x, a2y)

        m0 = plsc.load_gather(mm_v, [jnp.zeros((L,), jnp.int32)])
        inv_m0 = 1.0 / m0
        mi = mm_v[pl.ds(base, L)]
        ox_v[...] = a2x * (G_CONST * mi)
        oy_v[...] = a2y * (G_CONST * mi)
        opx_v[...] = px_v[...] * inv_m0
        opy_v[...] = py_v[...] * inv_m0

        pltpu.sync_copy(opx_v, opx_h.at[pl.ds(obase, ROWS)])
        pltpu.sync_copy(opy_v, opy_h.at[pl.ds(obase, ROWS)])
        pltpu.sync_copy(ox_v, ox_h.at[pl.ds(obase, ROWS)])
        pltpu.sync_copy(oy_v, oy_h.at[pl.ds(obase, ROWS)])


_f32 = jnp.float32
_svec = jax.ShapeDtypeStruct((T_SC,), _f32)

_nbody_sc = functools.partial(
    pl.kernel,
    out_type=(_svec, _svec, _svec, _svec),
    mesh=plsc.VectorSubcoreMesh(core_axis_name="c", subcore_axis_name="s"),
    compiler_params=pltpu.CompilerParams(needs_layout_passes=False),
    scratch_types=[
        pltpu.VMEM((N,), _f32),
        pltpu.VMEM((N,), _f32),
        pltpu.VMEM((N,), _f32),
        pltpu.VMEM((ROWS,), _f32),
        pltpu.VMEM((ROWS,), _f32),
        pltpu.VMEM((ROWS,), _f32),
        pltpu.VMEM((ROWS,), _f32),
        pltpu.VMEM((ROWS,), _f32),
        pltpu.VMEM((ROWS,), _f32),
        pltpu.VMEM((L,), _f32),
        pltpu.VMEM((L,), _f32),
        pltpu.VMEM((L,), _f32),
        pltpu.VMEM((L,), _f32),
        pltpu.VMEM_SHARED((GPC, 2, L), _f32),
        pltpu.VMEM_SHARED((GPC, 2, L), _f32),
    ],
)(_sc_body)


def _tc_body(xb_ref, xf_ref, xt_ref, mc_ref, mf_ref, mr_ref, out_ref):
    i0 = pl.program_id(0) * BI
    xb = xb_ref[...]
    qix = xb[:, 0:1]
    qiy = xb[:, 1:2]
    qkx = xt_ref[0:1, :]
    qky = xt_ref[1:2, :]
    mk = mr_ref[...]
    dx = qix - qkx
    dy = qiy - qky
    d2 = jnp.maximum(dx * dx + dy * dy, 1e-24)
    y = lax.rsqrt(d2)
    w = y * y * y * mk
    accx = jnp.sum(w * dx, axis=1, keepdims=True)
    accy = jnp.sum(w * dy, axis=1, keepdims=True)

    prow = xf_ref[pl.ds(jnp.maximum(i0 - 1, 0), 1), :]
    nrow = xf_ref[pl.ds(jnp.minimum(i0 + BI, N - 1), 1), :]
    pm = mf_ref[pl.ds(jnp.maximum(i0 - 1, 0), 1), :]
    nm = mf_ref[pl.ds(jnp.minimum(i0 + BI, N - 1), 1), :]
    mcol = mc_ref[...]
    prev_x = jnp.concatenate([prow, xb[: BI - 1, :]], axis=0)
    next_x = jnp.concatenate([xb[1:, :], nrow], axis=0)
    prev_m = jnp.concatenate([pm, mcol[: BI - 1, :]], axis=0)
    next_m = jnp.concatenate([mcol[1:, :], nm], axis=0)
    for xn, mn in ((prev_x, prev_m), (next_x, next_m)):
        ddx = qix - xn[:, 0:1]
        ddy = qiy - xn[:, 1:2]
        dd2 = jnp.maximum(ddx * ddx + ddy * ddy, 1e-24)
        yy = lax.rsqrt(dd2)
        ww = yy * yy * yy * mn
        accx = accx + ww * ddx
        accy = accy + ww * ddy

    mi = mcol * G_CONST
    inv0 = 1.0 / mr_ref[0, 0]
    out_ref[...] = jnp.concatenate(
        [xb[:, 2:3] * inv0, xb[:, 3:4] * inv0, accx * mi, accy * mi], axis=1)


_nbody_tc = pl.pallas_call(
    _tc_body,
    grid=(S_TC // BI,),
    in_specs=[
        pl.BlockSpec((BI, 4), lambda i: (i, 0)),
        pl.BlockSpec((N, 4), lambda i: (0, 0)),
        pl.BlockSpec((4, N), lambda i: (0, 0)),
        pl.BlockSpec((BI, 1), lambda i: (i, 0)),
        pl.BlockSpec((N, 1), lambda i: (0, 0)),
        pl.BlockSpec((1, N), lambda i: (0, 0)),
    ],
    out_specs=pl.BlockSpec((BI, 4), lambda i: (i, 0)),
    out_shape=jax.ShapeDtypeStruct((S_TC, 4), _f32),
)


@jax.jit
def _run(x, m):
    xt = x.T
    mm = m.reshape(N)
    mr = m.reshape(1, N)

    tc_out = _nbody_tc(x, x, xt, m, m, mr)
    opx, opy, ox, oy = _nbody_sc(xt, mm)
    sc_out = jnp.stack([opx, opy, ox, oy], axis=-1)
    return jnp.concatenate([tc_out, sc_out], axis=0)


def kernel(t, x, m):
    del t
    return _run(x, m)
